# baseline (speedup 1.0000x reference)
import jax
import jax.numpy as jnp
from jax import lax
from jax.experimental import pallas as pl
from jax.experimental.pallas import tpu as pltpu
from jax.experimental.pallas import tpu_sc as plsc

B = 16384
EMB = 128
NC = 2
NS = 16
NW = NC * NS
BPW = B // NW
CH = 128
NCH = BPW // CH
GRP = CH // 16


def _mf_body(uid_hbm, iid_hbm, w_hbm, h_hbm, out_hbm,
             uid_v, iid_v, wb0, wb1, hb0, hb1, out_v,
             sw0, sw1, sh0, sh1, sidx, swb, shb):
    wid = lax.axis_index("s") * NC + lax.axis_index("c")
    base = wid * BPW

    cu = pltpu.async_copy(uid_hbm.at[pl.ds(base, BPW)], uid_v, sidx)
    ci = pltpu.async_copy(iid_hbm.at[pl.ds(base, BPW)], iid_v, sidx)
    cu.wait()
    ci.wait()

    wbufs = (wb0, wb1)
    hbufs = (hb0, hb1)
    wsems = (sw0, sw1)
    hsems = (sh0, sh1)

    def start(c):
        slot = c % 2
        cw = pltpu.async_copy(
            w_hbm.at[uid_v.at[pl.ds(c * CH, CH)]], wbufs[slot], wsems[slot])
        chh = pltpu.async_copy(
            h_hbm.at[iid_v.at[pl.ds(c * CH, CH)]], hbufs[slot], hsems[slot])
        return cw, chh

    lane = lax.iota(jnp.int32, 16)
    butterfly_perms = [lane ^ s for s in (8, 4, 2, 1)]
    gmode = "promise_in_bounds"

    HF = 32

    def start_half(off, n, wsem, hsem):
        rows = pl.ds(off, n)
        cw = pltpu.async_copy(w_hbm.at[uid_v.at[rows]], wb0.at[rows], wsem)
        chh = pltpu.async_copy(h_hbm.at[iid_v.at[rows]], hb0.at[rows], hsem)
        return cw, chh

    half_a = start_half(0, HF, sw0, sh0)
    half_b = start_half(HF, CH - HF, swb, shb)
    inflight = {1: start(1)}
    halves = {0: (half_a, half_b)}
    for c in range(NCH):
        if c >= 1 and c + 1 < NCH:
            inflight[c + 1] = start(c + 1)
        if c in halves:
            for h in halves[c][0]:
                h.wait()
        else:
            for h in inflight.pop(c):
                h.wait()
        slot = c % 2
        wref = wbufs[slot]
        href = hbufs[slot]

        def group_body(g, _, wref=wref, href=href, c=c):
            row0 = g * 16

            def row_body(r, res):
                row = row0 + r
                ps = []
                for j in range(EMB // 16):
                    w = wref[row, pl.ds(j * 16, 16)]
                    h = href[row, pl.ds(j * 16, 16)]
                    ps.append(w * h)
                while len(ps) > 1:
                    ps = [a + b for a, b in zip(ps[0::2], ps[1::2])]
                acc = ps[0]
                for perm in butterfly_perms:
                    acc = acc + acc.at[perm].get(mode=gmode)
                return jnp.where(lane == r, acc, res)

            res = lax.fori_loop(0, 16, row_body, jnp.zeros((16,), jnp.float32))
            pred = 1.0 / (1.0 + jnp.exp(-res))
            out_v[pl.ds(c * CH + row0, 16)] = pred
            return 0

        if c in halves:
            lax.fori_loop(0, HF // 16, group_body, 0)
            for h in halves[c][1]:
                h.wait()
            lax.fori_loop(HF // 16, GRP, group_body, 0)
        else:
            lax.fori_loop(0, GRP, group_body, 0)

    pltpu.sync_copy(out_v, out_hbm.at[pl.ds(base, BPW)])


@jax.jit
def kernel(x, W, H):
    uidx = x[:, 0]
    iidx = x[:, 1]
    mesh = plsc.VectorSubcoreMesh(core_axis_name="c", subcore_axis_name="s")
    f = pl.kernel(
        _mf_body,
        out_type=jax.ShapeDtypeStruct((B,), jnp.float32),
        mesh=mesh,
        scratch_types=[
            pltpu.VMEM((BPW,), jnp.int32),
            pltpu.VMEM((BPW,), jnp.int32),
            pltpu.VMEM((CH, EMB), jnp.float32),
            pltpu.VMEM((CH, EMB), jnp.float32),
            pltpu.VMEM((CH, EMB), jnp.float32),
            pltpu.VMEM((CH, EMB), jnp.float32),
            pltpu.VMEM((BPW,), jnp.float32),
            pltpu.SemaphoreType.DMA,
            pltpu.SemaphoreType.DMA,
            pltpu.SemaphoreType.DMA,
            pltpu.SemaphoreType.DMA,
            pltpu.SemaphoreType.DMA,
            pltpu.SemaphoreType.DMA,
            pltpu.SemaphoreType.DMA,
        ],
    )
    return f(uidx, iidx, W, H)


# speedup vs baseline: 1.0276x; 1.0276x over previous
import jax
import jax.numpy as jnp
from jax import lax
from jax.experimental import pallas as pl
from jax.experimental.pallas import tpu as pltpu
from jax.experimental.pallas import tpu_sc as plsc

B = 16384
EMB = 128
NC = 2
NS = 16
NW = NC * NS
BPW = B // NW
CH = 128
NCH = BPW // CH
GRP = CH // 16


def _mf_body(uid_hbm, iid_hbm, w_hbm, h_hbm, out_hbm,
             uid_v, iid_v, wb0, wb1, hb0, hb1, out_v,
             sw0, sw1, sh0, sh1, sidx, swb, shb):
    wid = lax.axis_index("s") * NC + lax.axis_index("c")
    base = wid * BPW

    cu = pltpu.async_copy(uid_hbm.at[pl.ds(base, BPW)], uid_v, sidx)
    ci = pltpu.async_copy(iid_hbm.at[pl.ds(base, BPW)], iid_v, sidx)
    cu.wait()
    ci.wait()

    wbufs = (wb0, wb1)
    hbufs = (hb0, hb1)
    wsems = (sw0, sw1)
    hsems = (sh0, sh1)

    def start(c):
        slot = c % 2
        cw = pltpu.async_copy(
            w_hbm.at[uid_v.at[pl.ds(c * CH, CH)]], wbufs[slot], wsems[slot])
        chh = pltpu.async_copy(
            h_hbm.at[iid_v.at[pl.ds(c * CH, CH)]], hbufs[slot], hsems[slot])
        return cw, chh

    lane = lax.iota(jnp.int32, 16)
    butterfly_perms = [lane ^ s for s in (8, 4, 2, 1)]
    gmode = "promise_in_bounds"

    HF = 64

    def start_half(off, n, wsem, hsem):
        rows = pl.ds(off, n)
        cw = pltpu.async_copy(w_hbm.at[uid_v.at[rows]], wb0.at[rows], wsem)
        chh = pltpu.async_copy(h_hbm.at[iid_v.at[rows]], hb0.at[rows], hsem)
        return cw, chh

    half_a = start_half(0, HF, sw0, sh0)
    half_b = start_half(HF, CH - HF, swb, shb)
    out_copies = []
    inflight = {1: start(1)}
    halves = {0: (half_a, half_b)}
    for c in range(NCH):
        if c >= 1 and c + 1 < NCH:
            inflight[c + 1] = start(c + 1)
        if c in halves:
            for h in halves[c][0]:
                h.wait()
        else:
            for h in inflight.pop(c):
                h.wait()
        slot = c % 2
        wref = wbufs[slot]
        href = hbufs[slot]

        def group_body(g, _, wref=wref, href=href, c=c):
            row0 = g * 16

            def row_body(r, res):
                row = row0 + r
                ps = []
                for j in range(EMB // 16):
                    w = wref[row, pl.ds(j * 16, 16)]
                    h = href[row, pl.ds(j * 16, 16)]
                    ps.append(w * h)
                while len(ps) > 1:
                    ps = [a + b for a, b in zip(ps[0::2], ps[1::2])]
                acc = ps[0]
                for perm in butterfly_perms:
                    acc = acc + acc.at[perm].get(mode=gmode)
                return jnp.where(lane == r, acc, res)

            res = lax.fori_loop(0, 16, row_body, jnp.zeros((16,), jnp.float32))
            pred = 1.0 / (1.0 + jnp.exp(-res))
            out_v[pl.ds(c * CH + row0, 16)] = pred
            return 0

        if c in halves:
            lax.fori_loop(0, HF // 16, group_body, 0)
            for h in halves[c][1]:
                h.wait()
            lax.fori_loop(HF // 16, GRP, group_body, 0)
        else:
            lax.fori_loop(0, GRP, group_body, 0)

        out_copies.append(pltpu.async_copy(
            out_v.at[pl.ds(c * CH, CH)],
            out_hbm.at[pl.ds(base + c * CH, CH)], sidx))

    for oc in out_copies:
        oc.wait()


@jax.jit
def kernel(x, W, H):
    uidx = x[:, 0]
    iidx = x[:, 1]
    mesh = plsc.VectorSubcoreMesh(core_axis_name="c", subcore_axis_name="s")
    f = pl.kernel(
        _mf_body,
        out_type=jax.ShapeDtypeStruct((B,), jnp.float32),
        mesh=mesh,
        scratch_types=[
            pltpu.VMEM((BPW,), jnp.int32),
            pltpu.VMEM((BPW,), jnp.int32),
            pltpu.VMEM((CH, EMB), jnp.float32),
            pltpu.VMEM((CH, EMB), jnp.float32),
            pltpu.VMEM((CH, EMB), jnp.float32),
            pltpu.VMEM((CH, EMB), jnp.float32),
            pltpu.VMEM((BPW,), jnp.float32),
            pltpu.SemaphoreType.DMA,
            pltpu.SemaphoreType.DMA,
            pltpu.SemaphoreType.DMA,
            pltpu.SemaphoreType.DMA,
            pltpu.SemaphoreType.DMA,
            pltpu.SemaphoreType.DMA,
            pltpu.SemaphoreType.DMA,
        ],
    )
    return f(uidx, iidx, W, H)
